# Initial kernel scaffold; baseline (speedup 1.0000x reference)
import jax
import jax.numpy as jnp
from jax.experimental import pallas as pl


def kernel(x, kohonen_weights, grossberg_weights):
    raise NotImplementedError("write your pallas kernel here")


# speedup vs baseline: 6.4288x; 6.4288x over previous
import functools

import jax
import jax.numpy as jnp
from jax import lax
from jax.experimental import pallas as pl
from jax.experimental.pallas import tpu as pltpu
from jax.experimental.pallas import tpu_sc as plsc

BATCH = 4096
INPUT_SIZE = 256
HIDDEN_SIZE = 8192
OUTPUT_SIZE = 256

H_BLK = 1024
N_HBLK = HIDDEN_SIZE // H_BLK


def _argmin_body(x_ref, w_ref, val_ref, idx_ref):
    h = pl.program_id(0)
    x = x_ref[...]
    w = w_ref[...]
    x_sq = jnp.sum(x * x, axis=1, keepdims=True)
    w_sq = jnp.sum(w * w, axis=1)[None, :]
    p = lax.dot_general(x, w, (((1,), (1,)), ((), ())),
                        preferred_element_type=jnp.float32)
    d2 = (x_sq + w_sq) - 2.0 * p
    dist = jnp.sqrt(jnp.clip(d2, 0.0, None))
    rowmin = jnp.min(dist, axis=1, keepdims=True)
    col = lax.broadcasted_iota(jnp.int32, dist.shape, 1) + h * H_BLK
    rowidx = jnp.min(jnp.where(dist == rowmin, col, jnp.int32(2**30)),
                     axis=1, keepdims=True)

    @pl.when(h == 0)
    def _init():
        val_ref[...] = rowmin
        idx_ref[...] = rowidx

    @pl.when(h != 0)
    def _update():
        prev_val = val_ref[...]
        prev_idx = idx_ref[...]
        upd = rowmin < prev_val
        val_ref[...] = jnp.where(upd, rowmin, prev_val)
        idx_ref[...] = jnp.where(upd, rowidx, prev_idx)


def _winners(x, kohonen_weights):
    _, idx = pl.pallas_call(
        _argmin_body,
        grid=(N_HBLK,),
        in_specs=[
            pl.BlockSpec((BATCH, INPUT_SIZE), lambda h: (0, 0)),
            pl.BlockSpec((H_BLK, INPUT_SIZE), lambda h: (h, 0)),
        ],
        out_specs=[
            pl.BlockSpec((BATCH, 1), lambda h: (0, 0)),
            pl.BlockSpec((BATCH, 1), lambda h: (0, 0)),
        ],
        out_shape=[
            jax.ShapeDtypeStruct((BATCH, 1), jnp.float32),
            jax.ShapeDtypeStruct((BATCH, 1), jnp.int32),
        ],
    )(x, kohonen_weights)
    return idx.reshape(BATCH)


def _sc_gather(gt, winners):
    info = plsc.get_sparse_core_info()
    nc, ns = info.num_cores, info.num_subcores
    nw = nc * ns
    b_per_w = BATCH // nw
    mesh = plsc.VectorSubcoreMesh(core_axis_name="c", subcore_axis_name="s")

    @functools.partial(
        pl.kernel, mesh=mesh,
        out_type=jax.ShapeDtypeStruct((BATCH, OUTPUT_SIZE), jnp.float32),
        scratch_types=[
            pltpu.VMEM((b_per_w,), jnp.int32),
            pltpu.VMEM((b_per_w, OUTPUT_SIZE), jnp.float32),
            pltpu.SemaphoreType.DMA,
        ],
    )
    def gather_kernel(gt_hbm, idx_hbm, out_hbm, idx_v, rows_v, sem):
        wid = lax.axis_index("s") * nc + lax.axis_index("c")
        base = wid * b_per_w
        pltpu.sync_copy(idx_hbm.at[pl.ds(base, b_per_w)], idx_v)
        pltpu.async_copy(gt_hbm.at[idx_v], rows_v, sem).wait()
        pltpu.sync_copy(rows_v, out_hbm.at[pl.ds(base, b_per_w)])

    return gather_kernel(gt, winners)


def kernel(x, kohonen_weights, grossberg_weights):
    winners = _winners(x, kohonen_weights)
    gt = grossberg_weights.T
    output = _sc_gather(gt, winners)
    return (output, winners)


# speedup vs baseline: 7.1192x; 1.1074x over previous
import functools

import jax
import jax.numpy as jnp
from jax import lax
from jax.experimental import pallas as pl
from jax.experimental.pallas import tpu as pltpu
from jax.experimental.pallas import tpu_sc as plsc

BATCH = 4096
INPUT_SIZE = 256
HIDDEN_SIZE = 8192
OUTPUT_SIZE = 256

H_BLK = 1024
N_HBLK = HIDDEN_SIZE // H_BLK


def _argmin_body(x2_ref, w_ref, xsq_ref, wsq_ref, val_ref, idx_ref):
    h = pl.program_id(0)
    x2 = x2_ref[...]
    w = w_ref[...]
    x_sq = xsq_ref[...]
    w_sq = wsq_ref[...]
    p2 = lax.dot_general(x2, w, (((1,), (1,)), ((), ())),
                         preferred_element_type=jnp.float32)
    d2 = (x_sq + w_sq) - p2
    rmin = jnp.min(d2, axis=1, keepdims=True)
    rminc = jnp.maximum(rmin, 0.0)
    s = jnp.sqrt(rminc)
    q = s * s
    c1 = lax.bitcast_convert_type(
        lax.bitcast_convert_type(q, jnp.int32) + 1, jnp.float32)
    T = jnp.where(jnp.sqrt(c1) == s, c1, q)
    T = jnp.maximum(T, rminc)
    col = lax.broadcasted_iota(jnp.int32, d2.shape, 1)
    rowidx = jnp.min(jnp.where(d2 <= T, col, jnp.int32(2**30)),
                     axis=1, keepdims=True) + h * H_BLK

    @pl.when(h == 0)
    def _init():
        val_ref[...] = s
        idx_ref[...] = rowidx

    @pl.when(h != 0)
    def _update():
        prev_val = val_ref[...]
        prev_idx = idx_ref[...]
        upd = s < prev_val
        val_ref[...] = jnp.where(upd, s, prev_val)
        idx_ref[...] = jnp.where(upd, rowidx, prev_idx)


def _winners(x, kohonen_weights):
    x2 = x + x
    x_sq = jnp.sum(x * x, axis=1, keepdims=True)
    w_sq = jnp.sum(kohonen_weights * kohonen_weights, axis=1)[None, :]
    _, idx = pl.pallas_call(
        _argmin_body,
        grid=(N_HBLK,),
        in_specs=[
            pl.BlockSpec((BATCH, INPUT_SIZE), lambda h: (0, 0)),
            pl.BlockSpec((H_BLK, INPUT_SIZE), lambda h: (h, 0)),
            pl.BlockSpec((BATCH, 1), lambda h: (0, 0)),
            pl.BlockSpec((1, H_BLK), lambda h: (0, h)),
        ],
        out_specs=[
            pl.BlockSpec((BATCH, 1), lambda h: (0, 0)),
            pl.BlockSpec((BATCH, 1), lambda h: (0, 0)),
        ],
        out_shape=[
            jax.ShapeDtypeStruct((BATCH, 1), jnp.float32),
            jax.ShapeDtypeStruct((BATCH, 1), jnp.int32),
        ],
    )(x2, kohonen_weights, x_sq, w_sq)
    return idx.reshape(BATCH)


def _sc_gather(gt, winners):
    info = plsc.get_sparse_core_info()
    nc, ns = info.num_cores, info.num_subcores
    nw = nc * ns
    b_per_w = BATCH // nw
    mesh = plsc.VectorSubcoreMesh(core_axis_name="c", subcore_axis_name="s")

    @functools.partial(
        pl.kernel, mesh=mesh,
        out_type=jax.ShapeDtypeStruct((BATCH, OUTPUT_SIZE), jnp.float32),
        scratch_types=[
            pltpu.VMEM((b_per_w,), jnp.int32),
            pltpu.VMEM((b_per_w, OUTPUT_SIZE), jnp.float32),
            pltpu.SemaphoreType.DMA,
        ],
    )
    def gather_kernel(gt_hbm, idx_hbm, out_hbm, idx_v, rows_v, sem):
        wid = lax.axis_index("s") * nc + lax.axis_index("c")
        base = wid * b_per_w
        pltpu.sync_copy(idx_hbm.at[pl.ds(base, b_per_w)], idx_v)
        pltpu.async_copy(gt_hbm.at[idx_v], rows_v, sem).wait()
        pltpu.sync_copy(rows_v, out_hbm.at[pl.ds(base, b_per_w)])

    return gather_kernel(gt, winners)


def kernel(x, kohonen_weights, grossberg_weights):
    winners = _winners(x, kohonen_weights)
    gt = grossberg_weights.T
    output = _sc_gather(gt, winners)
    return (output, winners)


# speedup vs baseline: 7.6750x; 1.0781x over previous
import functools

import jax
import jax.numpy as jnp
from jax import lax
from jax.experimental import pallas as pl
from jax.experimental.pallas import tpu as pltpu
from jax.experimental.pallas import tpu_sc as plsc

BATCH = 4096
INPUT_SIZE = 256
HIDDEN_SIZE = 8192
OUTPUT_SIZE = 256

H_BLK = 1024
N_HBLK = HIDDEN_SIZE // H_BLK


def _argmin_body(x2_ref, w_ref, xsq_ref, wsq_ref, col_ref, val_ref, idx_ref):
    h = pl.program_id(0)
    x2 = x2_ref[...]
    w = w_ref[...]
    x_sq = xsq_ref[...]
    w_sq = wsq_ref[...]
    col = col_ref[...]
    p2 = lax.dot_general(x2, w, (((1,), (1,)), ((), ())),
                         preferred_element_type=jnp.float32)
    d2 = (x_sq + w_sq) - p2
    rmin = jnp.min(d2, axis=1, keepdims=True)
    rminc = jnp.maximum(rmin, 0.0)
    s = jnp.sqrt(rminc)
    q = s * s
    c1 = lax.bitcast_convert_type(
        lax.bitcast_convert_type(q, jnp.int32) + 1, jnp.float32)
    T = jnp.where(jnp.sqrt(c1) == s, c1, q)
    T = jnp.maximum(T, rminc)
    rowidx_f = jnp.min(jnp.where(d2 <= T, col, jnp.float32(jnp.inf)),
                       axis=1, keepdims=True)
    rowidx = rowidx_f.astype(jnp.int32) + h * H_BLK

    @pl.when(h == 0)
    def _init():
        val_ref[...] = s
        idx_ref[...] = rowidx

    @pl.when(h != 0)
    def _update():
        prev_val = val_ref[...]
        prev_idx = idx_ref[...]
        upd = s < prev_val
        val_ref[...] = jnp.where(upd, s, prev_val)
        idx_ref[...] = jnp.where(upd, rowidx, prev_idx)


def _winners(x, kohonen_weights):
    x2 = x + x
    x_sq = jnp.sum(x * x, axis=1, keepdims=True)
    w_sq = jnp.sum(kohonen_weights * kohonen_weights, axis=1)[None, :]
    col = jnp.arange(H_BLK, dtype=jnp.float32)[None, :]
    _, idx = pl.pallas_call(
        _argmin_body,
        grid=(N_HBLK,),
        in_specs=[
            pl.BlockSpec((BATCH, INPUT_SIZE), lambda h: (0, 0)),
            pl.BlockSpec((H_BLK, INPUT_SIZE), lambda h: (h, 0)),
            pl.BlockSpec((BATCH, 1), lambda h: (0, 0)),
            pl.BlockSpec((1, H_BLK), lambda h: (0, h)),
            pl.BlockSpec((1, H_BLK), lambda h: (0, 0)),
        ],
        out_specs=[
            pl.BlockSpec((BATCH, 1), lambda h: (0, 0)),
            pl.BlockSpec((BATCH, 1), lambda h: (0, 0)),
        ],
        out_shape=[
            jax.ShapeDtypeStruct((BATCH, 1), jnp.float32),
            jax.ShapeDtypeStruct((BATCH, 1), jnp.int32),
        ],
    )(x2, kohonen_weights, x_sq, w_sq, col)
    return idx.reshape(BATCH)


def _sc_gather(gt, winners):
    info = plsc.get_sparse_core_info()
    nc, ns = info.num_cores, info.num_subcores
    nw = nc * ns
    b_per_w = BATCH // nw
    mesh = plsc.VectorSubcoreMesh(core_axis_name="c", subcore_axis_name="s")

    @functools.partial(
        pl.kernel, mesh=mesh,
        out_type=jax.ShapeDtypeStruct((BATCH, OUTPUT_SIZE), jnp.float32),
        scratch_types=[
            pltpu.VMEM((b_per_w,), jnp.int32),
            pltpu.VMEM((b_per_w, OUTPUT_SIZE), jnp.float32),
            pltpu.SemaphoreType.DMA,
        ],
    )
    def gather_kernel(gt_hbm, idx_hbm, out_hbm, idx_v, rows_v, sem):
        wid = lax.axis_index("s") * nc + lax.axis_index("c")
        base = wid * b_per_w
        pltpu.sync_copy(idx_hbm.at[pl.ds(base, b_per_w)], idx_v)
        pltpu.async_copy(gt_hbm.at[idx_v], rows_v, sem).wait()
        pltpu.sync_copy(rows_v, out_hbm.at[pl.ds(base, b_per_w)])

    return gather_kernel(gt, winners)


def kernel(x, kohonen_weights, grossberg_weights):
    winners = _winners(x, kohonen_weights)
    gt = grossberg_weights.T
    output = _sc_gather(gt, winners)
    return (output, winners)


# speedup vs baseline: 7.7002x; 1.0033x over previous
import functools

import jax
import jax.numpy as jnp
from jax import lax
from jax.experimental import pallas as pl
from jax.experimental.pallas import tpu as pltpu
from jax.experimental.pallas import tpu_sc as plsc

BATCH = 4096
INPUT_SIZE = 256
HIDDEN_SIZE = 8192
OUTPUT_SIZE = 256

H_BLK = 1024
N_HBLK = HIDDEN_SIZE // H_BLK


def _argmin_body(x2_ref, w_ref, xsq_ref, wsq_ref, val_ref, idx_ref):
    h = pl.program_id(0)
    x2 = x2_ref[...]
    w = w_ref[...]
    x_sq = xsq_ref[...]
    w_sq = wsq_ref[...]
    p2 = lax.dot_general(x2, w, (((1,), (1,)), ((), ())),
                         preferred_element_type=jnp.float32)
    d2 = (x_sq + w_sq) - p2
    rmin = jnp.min(d2, axis=1, keepdims=True)
    rminc = jnp.maximum(rmin, 0.0)
    s = jnp.sqrt(rminc)
    q = s * s
    c1 = lax.bitcast_convert_type(
        lax.bitcast_convert_type(q, jnp.int32) + 1, jnp.float32)
    T = jnp.where(jnp.sqrt(c1) == s, c1, q)
    T = jnp.maximum(T, rminc)
    colf = lax.broadcasted_iota(jnp.int32, d2.shape, 1).astype(jnp.float32)
    rowidx_f = jnp.min(jnp.where(d2 <= T, colf, jnp.float32(2.0**30)),
                       axis=1, keepdims=True)
    rowidx = rowidx_f.astype(jnp.int32) + h * H_BLK

    @pl.when(h == 0)
    def _init():
        val_ref[...] = s
        idx_ref[...] = rowidx

    @pl.when(h != 0)
    def _update():
        prev_val = val_ref[...]
        prev_idx = idx_ref[...]
        upd = s < prev_val
        val_ref[...] = jnp.where(upd, s, prev_val)
        idx_ref[...] = jnp.where(upd, rowidx, prev_idx)


def _winners(x, kohonen_weights):
    x2 = x + x
    x_sq = jnp.sum(x * x, axis=1, keepdims=True)
    w_sq = jnp.sum(kohonen_weights * kohonen_weights, axis=1)[None, :]
    _, idx = pl.pallas_call(
        _argmin_body,
        grid=(N_HBLK,),
        in_specs=[
            pl.BlockSpec((BATCH, INPUT_SIZE), lambda h: (0, 0)),
            pl.BlockSpec((H_BLK, INPUT_SIZE), lambda h: (h, 0)),
            pl.BlockSpec((BATCH, 1), lambda h: (0, 0)),
            pl.BlockSpec((1, H_BLK), lambda h: (0, h)),
        ],
        out_specs=[
            pl.BlockSpec((BATCH, 1), lambda h: (0, 0)),
            pl.BlockSpec((BATCH, 1), lambda h: (0, 0)),
        ],
        out_shape=[
            jax.ShapeDtypeStruct((BATCH, 1), jnp.float32),
            jax.ShapeDtypeStruct((BATCH, 1), jnp.int32),
        ],
    )(x2, kohonen_weights, x_sq, w_sq)
    return idx.reshape(BATCH)


def _sc_gather(gt, winners):
    info = plsc.get_sparse_core_info()
    nc, ns = info.num_cores, info.num_subcores
    nw = nc * ns
    b_per_w = BATCH // nw
    mesh = plsc.VectorSubcoreMesh(core_axis_name="c", subcore_axis_name="s")

    @functools.partial(
        pl.kernel, mesh=mesh,
        out_type=jax.ShapeDtypeStruct((BATCH, OUTPUT_SIZE), jnp.float32),
        scratch_types=[
            pltpu.VMEM((b_per_w,), jnp.int32),
            pltpu.VMEM((b_per_w, OUTPUT_SIZE), jnp.float32),
            pltpu.SemaphoreType.DMA,
        ],
    )
    def gather_kernel(gt_hbm, idx_hbm, out_hbm, idx_v, rows_v, sem):
        wid = lax.axis_index("s") * nc + lax.axis_index("c")
        base = wid * b_per_w
        pltpu.sync_copy(idx_hbm.at[pl.ds(base, b_per_w)], idx_v)
        pltpu.async_copy(gt_hbm.at[idx_v], rows_v, sem).wait()
        pltpu.sync_copy(rows_v, out_hbm.at[pl.ds(base, b_per_w)])

    return gather_kernel(gt, winners)


def kernel(x, kohonen_weights, grossberg_weights):
    winners = _winners(x, kohonen_weights)
    gt = grossberg_weights.T
    output = _sc_gather(gt, winners)
    return (output, winners)


# speedup vs baseline: 8.2981x; 1.0776x over previous
import functools

import jax
import jax.numpy as jnp
from jax import lax
from jax.experimental import pallas as pl
from jax.experimental.pallas import tpu as pltpu
from jax.experimental.pallas import tpu_sc as plsc

BATCH = 4096
INPUT_SIZE = 256
HIDDEN_SIZE = 8192
OUTPUT_SIZE = 256

H_BLK = 2048
N_HBLK = HIDDEN_SIZE // H_BLK


def _argmin_body(x2_ref, w_ref, xsq_ref, wsq_ref, val_ref, idx_ref):
    h = pl.program_id(0)
    x2 = x2_ref[...]
    w = w_ref[...]
    x_sq = xsq_ref[...]
    w_sq = wsq_ref[...]
    p2 = lax.dot_general(x2, w, (((1,), (1,)), ((), ())),
                         preferred_element_type=jnp.float32)
    d2 = (x_sq + w_sq) - p2
    rmin = jnp.min(d2, axis=1, keepdims=True)
    rminc = jnp.maximum(rmin, 0.0)
    s = jnp.sqrt(rminc)
    q = s * s
    c1 = lax.bitcast_convert_type(
        lax.bitcast_convert_type(q, jnp.int32) + 1, jnp.float32)
    T = jnp.where(jnp.sqrt(c1) == s, c1, q)
    T = jnp.maximum(T, rminc)
    colf = lax.broadcasted_iota(jnp.int32, d2.shape, 1).astype(jnp.float32)
    rowidx_f = jnp.min(jnp.where(d2 <= T, colf, jnp.float32(2.0**30)),
                       axis=1, keepdims=True)
    rowidx = rowidx_f.astype(jnp.int32) + h * H_BLK

    @pl.when(h == 0)
    def _init():
        val_ref[...] = s
        idx_ref[...] = rowidx

    @pl.when(h != 0)
    def _update():
        prev_val = val_ref[...]
        prev_idx = idx_ref[...]
        upd = s < prev_val
        val_ref[...] = jnp.where(upd, s, prev_val)
        idx_ref[...] = jnp.where(upd, rowidx, prev_idx)


def _winners(x, kohonen_weights):
    x2 = x + x
    x_sq = jnp.sum(x * x, axis=1, keepdims=True)
    w_sq = jnp.sum(kohonen_weights * kohonen_weights, axis=1)[None, :]
    _, idx = pl.pallas_call(
        _argmin_body,
        grid=(N_HBLK,),
        in_specs=[
            pl.BlockSpec((BATCH, INPUT_SIZE), lambda h: (0, 0)),
            pl.BlockSpec((H_BLK, INPUT_SIZE), lambda h: (h, 0)),
            pl.BlockSpec((BATCH, 1), lambda h: (0, 0)),
            pl.BlockSpec((1, H_BLK), lambda h: (0, h)),
        ],
        out_specs=[
            pl.BlockSpec((BATCH, 1), lambda h: (0, 0)),
            pl.BlockSpec((BATCH, 1), lambda h: (0, 0)),
        ],
        out_shape=[
            jax.ShapeDtypeStruct((BATCH, 1), jnp.float32),
            jax.ShapeDtypeStruct((BATCH, 1), jnp.int32),
        ],
    )(x2, kohonen_weights, x_sq, w_sq)
    return idx.reshape(BATCH)


def _sc_gather(gt, winners):
    info = plsc.get_sparse_core_info()
    nc, ns = info.num_cores, info.num_subcores
    nw = nc * ns
    b_per_w = BATCH // nw
    mesh = plsc.VectorSubcoreMesh(core_axis_name="c", subcore_axis_name="s")

    @functools.partial(
        pl.kernel, mesh=mesh,
        out_type=jax.ShapeDtypeStruct((BATCH, OUTPUT_SIZE), jnp.float32),
        scratch_types=[
            pltpu.VMEM((b_per_w,), jnp.int32),
            pltpu.VMEM((b_per_w, OUTPUT_SIZE), jnp.float32),
            pltpu.SemaphoreType.DMA,
        ],
    )
    def gather_kernel(gt_hbm, idx_hbm, out_hbm, idx_v, rows_v, sem):
        wid = lax.axis_index("s") * nc + lax.axis_index("c")
        base = wid * b_per_w
        pltpu.sync_copy(idx_hbm.at[pl.ds(base, b_per_w)], idx_v)
        pltpu.async_copy(gt_hbm.at[idx_v], rows_v, sem).wait()
        pltpu.sync_copy(rows_v, out_hbm.at[pl.ds(base, b_per_w)])

    return gather_kernel(gt, winners)


def kernel(x, kohonen_weights, grossberg_weights):
    winners = _winners(x, kohonen_weights)
    gt = grossberg_weights.T
    output = _sc_gather(gt, winners)
    return (output, winners)


# speedup vs baseline: 9.0039x; 1.0851x over previous
import functools

import jax
import jax.numpy as jnp
from jax import lax
from jax.experimental import pallas as pl
from jax.experimental.pallas import tpu as pltpu
from jax.experimental.pallas import tpu_sc as plsc

BATCH = 4096
INPUT_SIZE = 256
HIDDEN_SIZE = 8192
OUTPUT_SIZE = 256

H_BLK = 2048
N_HBLK = HIDDEN_SIZE // H_BLK


def _argmin_body(x2_ref, w_ref, xsq_ref, wsq_ref, g_ref, val_ref, idx_ref,
                 gt_ref):
    h = pl.program_id(0)
    x2 = x2_ref[...]
    w = w_ref[...]
    x_sq = xsq_ref[...]
    w_sq = wsq_ref[...]
    gt_ref[...] = g_ref[...].T
    p2 = lax.dot_general(x2, w, (((1,), (1,)), ((), ())),
                         preferred_element_type=jnp.float32)
    d2 = (x_sq + w_sq) - p2
    rmin = jnp.min(d2, axis=1, keepdims=True)
    rminc = jnp.maximum(rmin, 0.0)
    s = jnp.sqrt(rminc)
    q = s * s
    c1 = lax.bitcast_convert_type(
        lax.bitcast_convert_type(q, jnp.int32) + 1, jnp.float32)
    T = jnp.where(jnp.sqrt(c1) == s, c1, q)
    T = jnp.maximum(T, rminc)
    colf = lax.broadcasted_iota(jnp.int32, d2.shape, 1).astype(jnp.float32)
    rowidx_f = jnp.min(jnp.where(d2 <= T, colf, jnp.float32(2.0**30)),
                       axis=1, keepdims=True)
    rowidx = rowidx_f.astype(jnp.int32) + h * H_BLK

    @pl.when(h == 0)
    def _init():
        val_ref[...] = s
        idx_ref[...] = rowidx

    @pl.when(h != 0)
    def _update():
        prev_val = val_ref[...]
        prev_idx = idx_ref[...]
        upd = s < prev_val
        val_ref[...] = jnp.where(upd, s, prev_val)
        idx_ref[...] = jnp.where(upd, rowidx, prev_idx)


def _winners(x, kohonen_weights, grossberg_weights):
    x2 = x + x
    x_sq = jnp.sum(x * x, axis=1, keepdims=True)
    w_sq = jnp.sum(kohonen_weights * kohonen_weights, axis=1)[None, :]
    _, idx, gt = pl.pallas_call(
        _argmin_body,
        grid=(N_HBLK,),
        in_specs=[
            pl.BlockSpec((BATCH, INPUT_SIZE), lambda h: (0, 0)),
            pl.BlockSpec((H_BLK, INPUT_SIZE), lambda h: (h, 0)),
            pl.BlockSpec((BATCH, 1), lambda h: (0, 0)),
            pl.BlockSpec((1, H_BLK), lambda h: (0, h)),
            pl.BlockSpec((OUTPUT_SIZE, H_BLK), lambda h: (0, h)),
        ],
        out_specs=[
            pl.BlockSpec((BATCH, 1), lambda h: (0, 0)),
            pl.BlockSpec((BATCH, 1), lambda h: (0, 0)),
            pl.BlockSpec((H_BLK, OUTPUT_SIZE), lambda h: (h, 0)),
        ],
        out_shape=[
            jax.ShapeDtypeStruct((BATCH, 1), jnp.float32),
            jax.ShapeDtypeStruct((BATCH, 1), jnp.int32),
            jax.ShapeDtypeStruct((HIDDEN_SIZE, OUTPUT_SIZE), jnp.float32),
        ],
    )(x2, kohonen_weights, x_sq, w_sq, grossberg_weights)
    return idx.reshape(BATCH), gt


def _sc_gather(gt, winners):
    info = plsc.get_sparse_core_info()
    nc, ns = info.num_cores, info.num_subcores
    nw = nc * ns
    b_per_w = BATCH // nw
    mesh = plsc.VectorSubcoreMesh(core_axis_name="c", subcore_axis_name="s")

    @functools.partial(
        pl.kernel, mesh=mesh,
        out_type=jax.ShapeDtypeStruct((BATCH, OUTPUT_SIZE), jnp.float32),
        scratch_types=[
            pltpu.VMEM((b_per_w,), jnp.int32),
            pltpu.VMEM((b_per_w, OUTPUT_SIZE), jnp.float32),
            pltpu.SemaphoreType.DMA,
        ],
    )
    def gather_kernel(gt_hbm, idx_hbm, out_hbm, idx_v, rows_v, sem):
        wid = lax.axis_index("s") * nc + lax.axis_index("c")
        base = wid * b_per_w
        pltpu.sync_copy(idx_hbm.at[pl.ds(base, b_per_w)], idx_v)
        pltpu.async_copy(gt_hbm.at[idx_v], rows_v, sem).wait()
        pltpu.sync_copy(rows_v, out_hbm.at[pl.ds(base, b_per_w)])

    return gather_kernel(gt, winners)


def kernel(x, kohonen_weights, grossberg_weights):
    winners, gt = _winners(x, kohonen_weights, grossberg_weights)
    output = _sc_gather(gt, winners)
    return (output, winners)


# speedup vs baseline: 9.0109x; 1.0008x over previous
import functools

import jax
import jax.numpy as jnp
from jax import lax
from jax.experimental import pallas as pl
from jax.experimental.pallas import tpu as pltpu
from jax.experimental.pallas import tpu_sc as plsc

BATCH = 4096
INPUT_SIZE = 256
HIDDEN_SIZE = 8192
OUTPUT_SIZE = 256

H_BLK = 2048
N_HBLK = HIDDEN_SIZE // H_BLK


def _argmin_body(x2_ref, w_ref, xsq_ref, wsq_ref, g_ref, val_ref, idx_ref,
                 gt_ref):
    h = pl.program_id(0)
    x2 = x2_ref[...]
    w = w_ref[...]
    x_sq = xsq_ref[...]
    w_sq = wsq_ref[...]
    gt_ref[...] = g_ref[...].T
    p2 = lax.dot_general(x2, w, (((1,), (1,)), ((), ())),
                         preferred_element_type=jnp.float32)
    d2 = (x_sq + w_sq) - p2
    rmin = jnp.min(d2, axis=1, keepdims=True)
    rminp = rmin.reshape(BATCH // 128, 128)
    rmincp = jnp.maximum(rminp, 0.0)
    sp = jnp.sqrt(rmincp)
    qp = sp * sp
    c1p = lax.bitcast_convert_type(
        lax.bitcast_convert_type(qp, jnp.int32) + 1, jnp.float32)
    Tp = jnp.where(jnp.sqrt(c1p) == sp, c1p, qp)
    Tp = jnp.maximum(Tp, rmincp)
    s = sp.reshape(BATCH, 1)
    T = Tp.reshape(BATCH, 1)
    colf = lax.broadcasted_iota(jnp.int32, d2.shape, 1).astype(jnp.float32)
    rowidx_f = jnp.min(jnp.where(d2 <= T, colf, jnp.float32(2.0**30)),
                       axis=1, keepdims=True)
    rowidx = rowidx_f.astype(jnp.int32) + h * H_BLK

    @pl.when(h == 0)
    def _init():
        val_ref[...] = s
        idx_ref[...] = rowidx

    @pl.when(h != 0)
    def _update():
        prev_val = val_ref[...]
        prev_idx = idx_ref[...]
        upd = s < prev_val
        val_ref[...] = jnp.where(upd, s, prev_val)
        idx_ref[...] = jnp.where(upd, rowidx, prev_idx)


def _winners(x, kohonen_weights, grossberg_weights):
    x2 = x + x
    x_sq = jnp.sum(x * x, axis=1, keepdims=True)
    w_sq = jnp.sum(kohonen_weights * kohonen_weights, axis=1)[None, :]
    _, idx, gt = pl.pallas_call(
        _argmin_body,
        grid=(N_HBLK,),
        in_specs=[
            pl.BlockSpec((BATCH, INPUT_SIZE), lambda h: (0, 0)),
            pl.BlockSpec((H_BLK, INPUT_SIZE), lambda h: (h, 0)),
            pl.BlockSpec((BATCH, 1), lambda h: (0, 0)),
            pl.BlockSpec((1, H_BLK), lambda h: (0, h)),
            pl.BlockSpec((OUTPUT_SIZE, H_BLK), lambda h: (0, h)),
        ],
        out_specs=[
            pl.BlockSpec((BATCH, 1), lambda h: (0, 0)),
            pl.BlockSpec((BATCH, 1), lambda h: (0, 0)),
            pl.BlockSpec((H_BLK, OUTPUT_SIZE), lambda h: (h, 0)),
        ],
        out_shape=[
            jax.ShapeDtypeStruct((BATCH, 1), jnp.float32),
            jax.ShapeDtypeStruct((BATCH, 1), jnp.int32),
            jax.ShapeDtypeStruct((HIDDEN_SIZE, OUTPUT_SIZE), jnp.float32),
        ],
    )(x2, kohonen_weights, x_sq, w_sq, grossberg_weights)
    return idx.reshape(BATCH), gt


def _sc_gather(gt, winners):
    info = plsc.get_sparse_core_info()
    nc, ns = info.num_cores, info.num_subcores
    nw = nc * ns
    b_per_w = BATCH // nw
    mesh = plsc.VectorSubcoreMesh(core_axis_name="c", subcore_axis_name="s")

    @functools.partial(
        pl.kernel, mesh=mesh,
        out_type=jax.ShapeDtypeStruct((BATCH, OUTPUT_SIZE), jnp.float32),
        scratch_types=[
            pltpu.VMEM((b_per_w,), jnp.int32),
            pltpu.VMEM((b_per_w, OUTPUT_SIZE), jnp.float32),
            pltpu.SemaphoreType.DMA,
        ],
    )
    def gather_kernel(gt_hbm, idx_hbm, out_hbm, idx_v, rows_v, sem):
        wid = lax.axis_index("s") * nc + lax.axis_index("c")
        base = wid * b_per_w
        pltpu.sync_copy(idx_hbm.at[pl.ds(base, b_per_w)], idx_v)
        pltpu.async_copy(gt_hbm.at[idx_v], rows_v, sem).wait()
        pltpu.sync_copy(rows_v, out_hbm.at[pl.ds(base, b_per_w)])

    return gather_kernel(gt, winners)


def kernel(x, kohonen_weights, grossberg_weights):
    winners, gt = _winners(x, kohonen_weights, grossberg_weights)
    output = _sc_gather(gt, winners)
    return (output, winners)


# speedup vs baseline: 9.0109x; 1.0000x over previous
import functools

import jax
import jax.numpy as jnp
from jax import lax
from jax.experimental import pallas as pl
from jax.experimental.pallas import tpu as pltpu
from jax.experimental.pallas import tpu_sc as plsc

BATCH = 4096
INPUT_SIZE = 256
HIDDEN_SIZE = 8192
OUTPUT_SIZE = 256

H_BLK = 2048
N_HBLK = HIDDEN_SIZE // H_BLK
B_BLK = 2048
N_BBLK = BATCH // B_BLK


def _argmin_body(x2_ref, w_ref, xsq_ref, wsq_ref, g_ref, val_ref, idx_ref,
                 gt_ref):
    b = pl.program_id(0)
    h = pl.program_id(1)
    x2 = x2_ref[...]
    w = w_ref[...]
    x_sq = xsq_ref[...]
    w_sq = wsq_ref[...]
    gt_ref[...] = g_ref[...].T
    p2 = lax.dot_general(x2, w, (((1,), (1,)), ((), ())),
                         preferred_element_type=jnp.float32)
    d2 = (x_sq + w_sq) - p2
    rmin = jnp.min(d2, axis=1, keepdims=True)
    rminp = rmin.reshape(B_BLK // 128, 128)
    rmincp = jnp.maximum(rminp, 0.0)
    sp = jnp.sqrt(rmincp)
    qp = sp * sp
    c1p = lax.bitcast_convert_type(
        lax.bitcast_convert_type(qp, jnp.int32) + 1, jnp.float32)
    Tp = jnp.where(jnp.sqrt(c1p) == sp, c1p, qp)
    Tp = jnp.maximum(Tp, rmincp)
    s = sp.reshape(B_BLK, 1)
    T = Tp.reshape(B_BLK, 1)
    colf = lax.broadcasted_iota(jnp.int32, d2.shape, 1).astype(jnp.float32)
    rowidx_f = jnp.min(jnp.where(d2 <= T, colf, jnp.float32(2.0**30)),
                       axis=1, keepdims=True)
    rowidx = rowidx_f.astype(jnp.int32) + h * H_BLK

    @pl.when(h == 0)
    def _init():
        val_ref[...] = s
        idx_ref[...] = rowidx

    @pl.when(h != 0)
    def _update():
        prev_val = val_ref[...]
        prev_idx = idx_ref[...]
        upd = s < prev_val
        val_ref[...] = jnp.where(upd, s, prev_val)
        idx_ref[...] = jnp.where(upd, rowidx, prev_idx)


def _winners(x, kohonen_weights, grossberg_weights):
    x2 = x + x
    x_sq = jnp.sum(x * x, axis=1, keepdims=True)
    w_sq = jnp.sum(kohonen_weights * kohonen_weights, axis=1)[None, :]
    _, idx, gt = pl.pallas_call(
        _argmin_body,
        grid=(N_BBLK, N_HBLK),
        in_specs=[
            pl.BlockSpec((B_BLK, INPUT_SIZE), lambda b, h: (b, 0)),
            pl.BlockSpec((H_BLK, INPUT_SIZE), lambda b, h: (h, 0)),
            pl.BlockSpec((B_BLK, 1), lambda b, h: (b, 0)),
            pl.BlockSpec((1, H_BLK), lambda b, h: (0, h)),
            pl.BlockSpec((OUTPUT_SIZE, H_BLK), lambda b, h: (0, h)),
        ],
        out_specs=[
            pl.BlockSpec((B_BLK, 1), lambda b, h: (b, 0)),
            pl.BlockSpec((B_BLK, 1), lambda b, h: (b, 0)),
            pl.BlockSpec((H_BLK, OUTPUT_SIZE), lambda b, h: (h, 0)),
        ],
        out_shape=[
            jax.ShapeDtypeStruct((BATCH, 1), jnp.float32),
            jax.ShapeDtypeStruct((BATCH, 1), jnp.int32),
            jax.ShapeDtypeStruct((HIDDEN_SIZE, OUTPUT_SIZE), jnp.float32),
        ],
    )(x2, kohonen_weights, x_sq, w_sq, grossberg_weights)
    return idx.reshape(BATCH), gt


def _sc_gather(gt, winners):
    info = plsc.get_sparse_core_info()
    nc, ns = info.num_cores, info.num_subcores
    nw = nc * ns
    b_per_w = BATCH // nw
    mesh = plsc.VectorSubcoreMesh(core_axis_name="c", subcore_axis_name="s")

    @functools.partial(
        pl.kernel, mesh=mesh,
        out_type=jax.ShapeDtypeStruct((BATCH, OUTPUT_SIZE), jnp.float32),
        scratch_types=[
            pltpu.VMEM((b_per_w,), jnp.int32),
            pltpu.VMEM((b_per_w, OUTPUT_SIZE), jnp.float32),
            pltpu.SemaphoreType.DMA,
        ],
    )
    def gather_kernel(gt_hbm, idx_hbm, out_hbm, idx_v, rows_v, sem):
        wid = lax.axis_index("s") * nc + lax.axis_index("c")
        base = wid * b_per_w
        pltpu.sync_copy(idx_hbm.at[pl.ds(base, b_per_w)], idx_v)
        pltpu.async_copy(gt_hbm.at[idx_v], rows_v, sem).wait()
        pltpu.sync_copy(rows_v, out_hbm.at[pl.ds(base, b_per_w)])

    return gather_kernel(gt, winners)


def kernel(x, kohonen_weights, grossberg_weights):
    winners, gt = _winners(x, kohonen_weights, grossberg_weights)
    output = _sc_gather(gt, winners)
    return (output, winners)
